# Initial kernel scaffold; baseline (speedup 1.0000x reference)
import jax
import jax.numpy as jnp
from jax.experimental import pallas as pl


def kernel(x, edge_features, edge_idx, We1, be1, We2, be2, Wn1, bn1, Wn2, bn2):
    raise NotImplementedError("write your pallas kernel here")


# speedup vs baseline: 3.1159x; 3.1159x over previous
import functools

import jax
import jax.numpy as jnp
from jax import lax
from jax.experimental import pallas as pl
from jax.experimental.pallas import tpu as pltpu
from jax.experimental.pallas import tpu_sc as plsc

N_NODES = 10000
N_EDGES = 320000
D = 128
D_EDGE = 16

NC = 2
NS = 16
NW = NC * NS
E_PER_W = N_EDGES // NW
K = 80
N_CH = E_PER_W // K
ROWS_PER_TILE = 624
ROWS_TAIL = N_NODES - NS * ROWS_PER_TILE

_mesh = plsc.VectorSubcoreMesh(core_axis_name="c", subcore_axis_name="s")
_f32 = jnp.float32


def _pre_body(x_ref, wa_ref, wb_ref, os_ref, ot_ref):
    xv = x_ref[...]
    os_ref[...] = jnp.dot(xv, wa_ref[...], preferred_element_type=_f32)
    ot_ref[...] = jnp.dot(xv, wb_ref[...], preferred_element_type=_f32)


@functools.partial(
    pl.kernel,
    out_type=(jax.ShapeDtypeStruct((N_EDGES, D), _f32),
              jax.ShapeDtypeStruct((N_EDGES, D), _f32)),
    mesh=_mesh,
    scratch_types=[
        pltpu.VMEM((K,), jnp.int32),
        pltpu.VMEM((K,), jnp.int32),
        pltpu.VMEM((K, D), _f32),
        pltpu.VMEM((K, D), _f32),
        pltpu.SemaphoreType.DMA,
        pltpu.SemaphoreType.DMA,
    ],
)
def _sc_gather(xs1_h, xt1_h, src_h, dst_h, os_h, ot_h,
               idx_s, idx_t, rows_s, rows_t, sem_s, sem_t):
    wid = lax.axis_index("s") * NC + lax.axis_index("c")
    base = wid * E_PER_W

    def chunk(c, carry):
        off = base + c * K
        pltpu.sync_copy(src_h.at[pl.ds(off, K)], idx_s)
        pltpu.sync_copy(dst_h.at[pl.ds(off, K)], idx_t)
        cs = pltpu.async_copy(xs1_h.at[idx_s], rows_s, sem_s)
        ct = pltpu.async_copy(xt1_h.at[idx_t], rows_t, sem_t)
        cs.wait()
        ct.wait()
        pltpu.sync_copy(rows_s, os_h.at[pl.ds(off, K)])
        pltpu.sync_copy(rows_t, ot_h.at[pl.ds(off, K)])
        return carry

    lax.fori_loop(0, N_CH, chunk, 0)


def _edge_mlp_body(ts_ref, tt_ref, ef_ref, w1e_ref, b1_ref, w2_ref, b2_ref,
                   out_ref):
    h = (ts_ref[...] + tt_ref[...]
         + jnp.dot(ef_ref[...], w1e_ref[...], preferred_element_type=_f32)
         + b1_ref[...])
    h = h * jax.nn.sigmoid(h)
    f = jnp.dot(h, w2_ref[...], preferred_element_type=_f32) + b2_ref[...]
    out_ref[...] = f * jax.nn.sigmoid(f)


@functools.partial(
    pl.kernel,
    out_type=jax.ShapeDtypeStruct((NC, N_NODES, D), _f32),
    mesh=_mesh,
    scratch_types=[
        pltpu.VMEM((K,), jnp.int32),
        pltpu.VMEM((K, D), _f32),
        pltpu.VMEM_SHARED((N_NODES, D), _f32),
    ],
)
def _sc_scatter(zeros_h, src_h, f_h, out_h, idx_v, rows_v, table):
    core = lax.axis_index("c")
    sid = lax.axis_index("s")
    wid = sid * NC + core
    row0 = sid * ROWS_PER_TILE
    pltpu.sync_copy(zeros_h.at[pl.ds(row0, ROWS_PER_TILE)],
                    table.at[pl.ds(row0, ROWS_PER_TILE)])
    tail0 = NS * ROWS_PER_TILE

    @pl.when(sid == 0)
    def _zero_tail():
        pltpu.sync_copy(zeros_h.at[pl.ds(tail0, ROWS_TAIL)],
                        table.at[pl.ds(tail0, ROWS_TAIL)])

    plsc.subcore_barrier()

    base = wid * E_PER_W

    def chunk(c, carry):
        off = base + c * K
        pltpu.sync_copy(src_h.at[pl.ds(off, K)], idx_v)
        pltpu.sync_copy(f_h.at[pl.ds(off, K)], rows_v)
        pltpu.sync_copy(rows_v, table.at[idx_v], add=True)
        return carry

    lax.fori_loop(0, N_CH, chunk, 0)
    plsc.subcore_barrier()
    pltpu.sync_copy(table.at[pl.ds(row0, ROWS_PER_TILE)],
                    out_h.at[core, pl.ds(row0, ROWS_PER_TILE)])

    @pl.when(sid == 0)
    def _write_tail():
        pltpu.sync_copy(table.at[pl.ds(tail0, ROWS_TAIL)],
                        out_h.at[core, pl.ds(tail0, ROWS_TAIL)])


def _node_mlp_body(x_ref, p_ref, w1x_ref, w1a_ref, b1_ref, w2_ref, b2_ref,
                   out_ref):
    agg = p_ref[0] + p_ref[1]
    h = (jnp.dot(x_ref[...], w1x_ref[...], preferred_element_type=_f32)
         + jnp.dot(agg, w1a_ref[...], preferred_element_type=_f32)
         + b1_ref[...])
    h = h * jax.nn.sigmoid(h)
    out_ref[...] = (jnp.dot(h, w2_ref[...], preferred_element_type=_f32)
                    + b2_ref[...])


def kernel(x, edge_features, edge_idx, We1, be1, We2, be2,
           Wn1, bn1, Wn2, bn2):
    src = edge_idx[0].astype(jnp.int32)
    dst = edge_idx[1].astype(jnp.int32)
    We1a = We1[:D]
    We1b = We1[D:2 * D]
    We1e = We1[2 * D:]
    Wn1x = Wn1[:D]
    Wn1a = Wn1[D:]
    b1e = be1.reshape(1, D)
    b2e = be2.reshape(1, D)
    b1n = bn1.reshape(1, D)
    b2n = bn2.reshape(1, D)
    zeros = jnp.zeros((N_NODES, D), _f32)

    xs1, xt1 = pl.pallas_call(
        _pre_body,
        out_shape=(jax.ShapeDtypeStruct((N_NODES, D), _f32),
                   jax.ShapeDtypeStruct((N_NODES, D), _f32)),
    )(x, We1a, We1b)

    tmp_s, tmp_t = _sc_gather(xs1, xt1, src, dst)

    E_BLK = 4000
    f = pl.pallas_call(
        _edge_mlp_body,
        grid=(N_EDGES // E_BLK,),
        in_specs=[
            pl.BlockSpec((E_BLK, D), lambda i: (i, 0)),
            pl.BlockSpec((E_BLK, D), lambda i: (i, 0)),
            pl.BlockSpec((E_BLK, D_EDGE), lambda i: (i, 0)),
            pl.BlockSpec((D_EDGE, D), lambda i: (0, 0)),
            pl.BlockSpec((1, D), lambda i: (0, 0)),
            pl.BlockSpec((D, D), lambda i: (0, 0)),
            pl.BlockSpec((1, D), lambda i: (0, 0)),
        ],
        out_specs=pl.BlockSpec((E_BLK, D), lambda i: (i, 0)),
        out_shape=jax.ShapeDtypeStruct((N_EDGES, D), _f32),
    )(tmp_s, tmp_t, edge_features, We1e, b1e, We2, b2e)

    partials = _sc_scatter(zeros, src, f)

    N_BLK = 2000
    out = pl.pallas_call(
        _node_mlp_body,
        grid=(N_NODES // N_BLK,),
        in_specs=[
            pl.BlockSpec((N_BLK, D), lambda i: (i, 0)),
            pl.BlockSpec((NC, N_BLK, D), lambda i: (0, i, 0)),
            pl.BlockSpec((D, D), lambda i: (0, 0)),
            pl.BlockSpec((D, D), lambda i: (0, 0)),
            pl.BlockSpec((1, D), lambda i: (0, 0)),
            pl.BlockSpec((D, D), lambda i: (0, 0)),
            pl.BlockSpec((1, D), lambda i: (0, 0)),
        ],
        out_specs=pl.BlockSpec((N_BLK, D), lambda i: (i, 0)),
        out_shape=jax.ShapeDtypeStruct((N_NODES, D), _f32),
    )(x, partials, Wn1x, Wn1a, b1n, Wn2, b2n)
    return out


# speedup vs baseline: 3.8868x; 1.2474x over previous
import functools

import jax
import jax.numpy as jnp
from jax import lax
from jax.experimental import pallas as pl
from jax.experimental.pallas import tpu as pltpu
from jax.experimental.pallas import tpu_sc as plsc

N_NODES = 10000
N_EDGES = 320000
D = 128
D_EDGE = 16

NC = 2
NS = 16
NW = NC * NS
E_PER_W = N_EDGES // NW
K = 40
N_CH = E_PER_W // K
NB = 2
N_GRP = N_CH // NB
ROWS_PER_TILE = 624
ROWS_TAIL = N_NODES - NS * ROWS_PER_TILE

_mesh = plsc.VectorSubcoreMesh(core_axis_name="c", subcore_axis_name="s")
_f32 = jnp.float32


def _pre_body(x_ref, wa_ref, wb_ref, os_ref, ot_ref):
    xv = x_ref[...]
    os_ref[...] = jnp.dot(xv, wa_ref[...], preferred_element_type=_f32)
    ot_ref[...] = jnp.dot(xv, wb_ref[...], preferred_element_type=_f32)


@functools.partial(
    pl.kernel,
    out_type=(jax.ShapeDtypeStruct((N_EDGES, D), _f32),
              jax.ShapeDtypeStruct((N_EDGES, D), _f32)),
    mesh=_mesh,
    scratch_types=[
        pltpu.VMEM((E_PER_W,), jnp.int32),
        pltpu.VMEM((E_PER_W,), jnp.int32),
        pltpu.VMEM((NB, K, D), _f32),
        pltpu.VMEM((NB, K, D), _f32),
        pltpu.SemaphoreType.DMA((NB,)),
        pltpu.SemaphoreType.DMA((NB,)),
    ],
)
def _sc_gather(xs1_h, xt1_h, src_h, dst_h, os_h, ot_h,
               idxs_all, idxt_all, rows_s, rows_t, gsem, wsem):
    wid = lax.axis_index("s") * NC + lax.axis_index("c")
    base = wid * E_PER_W
    pltpu.sync_copy(src_h.at[pl.ds(base, E_PER_W)], idxs_all)
    pltpu.sync_copy(dst_h.at[pl.ds(base, E_PER_W)], idxt_all)

    def group(g, carry):
        descs = []
        for b in range(NB):
            c = g * NB + b

            @pl.when(g > 0)
            def _drain_writeout(b=b):
                pltpu.make_async_copy(xs1_h.at[pl.ds(0, K)], rows_s.at[b],
                                      wsem.at[b]).wait()
                pltpu.make_async_copy(xs1_h.at[pl.ds(0, K)], rows_t.at[b],
                                      wsem.at[b]).wait()

            loff = c * K
            cs = pltpu.async_copy(xs1_h.at[idxs_all.at[pl.ds(loff, K)]],
                                  rows_s.at[b], gsem.at[b])
            ct = pltpu.async_copy(xt1_h.at[idxt_all.at[pl.ds(loff, K)]],
                                  rows_t.at[b], gsem.at[b])
            descs.append((cs, ct))
        for b in range(NB):
            c = g * NB + b
            off = base + c * K
            cs, ct = descs[b]
            cs.wait()
            ct.wait()
            pltpu.async_copy(rows_s.at[b], os_h.at[pl.ds(off, K)], wsem.at[b])
            pltpu.async_copy(rows_t.at[b], ot_h.at[pl.ds(off, K)], wsem.at[b])
        return carry

    lax.fori_loop(0, N_GRP, group, 0)
    for b in range(NB):
        pltpu.make_async_copy(xs1_h.at[pl.ds(0, K)], rows_s.at[b],
                              wsem.at[b]).wait()
        pltpu.make_async_copy(xs1_h.at[pl.ds(0, K)], rows_t.at[b],
                              wsem.at[b]).wait()


def _edge_mlp_body(ts_ref, tt_ref, ef_ref, w1e_ref, b1_ref, w2_ref, b2_ref,
                   out_ref):
    h = (ts_ref[...] + tt_ref[...]
         + jnp.dot(ef_ref[...], w1e_ref[...], preferred_element_type=_f32)
         + b1_ref[...])
    h = h * jax.nn.sigmoid(h)
    f = jnp.dot(h, w2_ref[...], preferred_element_type=_f32) + b2_ref[...]
    out_ref[...] = f * jax.nn.sigmoid(f)


@functools.partial(
    pl.kernel,
    out_type=jax.ShapeDtypeStruct((NC, N_NODES, D), _f32),
    mesh=_mesh,
    scratch_types=[
        pltpu.VMEM((N_CH, K), jnp.int32),
        pltpu.VMEM((NB, K, D), _f32),
        pltpu.SemaphoreType.DMA((NB,)),
        pltpu.SemaphoreType.DMA((NB,)),
        pltpu.VMEM_SHARED((N_NODES, D), _f32),
    ],
)
def _sc_scatter(zeros_h, src3_h, f_h, out_h, idx2d, rows, rsem, ssem, table):
    core = lax.axis_index("c")
    sid = lax.axis_index("s")
    wid = sid * NC + core
    pltpu.sync_copy(src3_h.at[wid], idx2d)
    row0 = sid * ROWS_PER_TILE
    pltpu.sync_copy(zeros_h.at[pl.ds(row0, ROWS_PER_TILE)],
                    table.at[pl.ds(row0, ROWS_PER_TILE)])
    tail0 = NS * ROWS_PER_TILE

    @pl.when(sid == 0)
    def _zero_tail():
        pltpu.sync_copy(zeros_h.at[pl.ds(tail0, ROWS_TAIL)],
                        table.at[pl.ds(tail0, ROWS_TAIL)])

    plsc.subcore_barrier()

    base = wid * E_PER_W

    def group(g, carry):
        descs = []
        for b in range(NB):
            c = g * NB + b

            @pl.when(g > 0)
            def _drain_scatter(b=b):
                pltpu.make_async_copy(f_h.at[pl.ds(0, K)], rows.at[b],
                                      ssem.at[b]).wait()

            off = base + c * K
            descs.append(pltpu.async_copy(f_h.at[pl.ds(off, K)], rows.at[b],
                                          rsem.at[b]))
        for b in range(NB):
            c = g * NB + b
            descs[b].wait()
            pltpu.async_copy(rows.at[b], table.at[idx2d.at[c]], ssem.at[b],
                             add=True)
        return carry

    lax.fori_loop(0, N_GRP, group, 0)
    for b in range(NB):
        pltpu.make_async_copy(f_h.at[pl.ds(0, K)], rows.at[b],
                              ssem.at[b]).wait()
    plsc.subcore_barrier()
    pltpu.sync_copy(table.at[pl.ds(row0, ROWS_PER_TILE)],
                    out_h.at[core, pl.ds(row0, ROWS_PER_TILE)])

    @pl.when(sid == 0)
    def _write_tail():
        pltpu.sync_copy(table.at[pl.ds(tail0, ROWS_TAIL)],
                        out_h.at[core, pl.ds(tail0, ROWS_TAIL)])


def _node_mlp_body(x_ref, p_ref, w1x_ref, w1a_ref, b1_ref, w2_ref, b2_ref,
                   out_ref):
    agg = p_ref[0] + p_ref[1]
    h = (jnp.dot(x_ref[...], w1x_ref[...], preferred_element_type=_f32)
         + jnp.dot(agg, w1a_ref[...], preferred_element_type=_f32)
         + b1_ref[...])
    h = h * jax.nn.sigmoid(h)
    out_ref[...] = (jnp.dot(h, w2_ref[...], preferred_element_type=_f32)
                    + b2_ref[...])


def kernel(x, edge_features, edge_idx, We1, be1, We2, be2,
           Wn1, bn1, Wn2, bn2):
    src = edge_idx[0].astype(jnp.int32)
    dst = edge_idx[1].astype(jnp.int32)
    We1a = We1[:D]
    We1b = We1[D:2 * D]
    We1e = We1[2 * D:]
    Wn1x = Wn1[:D]
    Wn1a = Wn1[D:]
    b1e = be1.reshape(1, D)
    b2e = be2.reshape(1, D)
    b1n = bn1.reshape(1, D)
    b2n = bn2.reshape(1, D)
    zeros = jnp.zeros((N_NODES, D), _f32)

    xs1, xt1 = pl.pallas_call(
        _pre_body,
        out_shape=(jax.ShapeDtypeStruct((N_NODES, D), _f32),
                   jax.ShapeDtypeStruct((N_NODES, D), _f32)),
    )(x, We1a, We1b)

    tmp_s, tmp_t = _sc_gather(xs1, xt1, src, dst)

    E_BLK = 4000
    f = pl.pallas_call(
        _edge_mlp_body,
        grid=(N_EDGES // E_BLK,),
        in_specs=[
            pl.BlockSpec((E_BLK, D), lambda i: (i, 0)),
            pl.BlockSpec((E_BLK, D), lambda i: (i, 0)),
            pl.BlockSpec((E_BLK, D_EDGE), lambda i: (i, 0)),
            pl.BlockSpec((D_EDGE, D), lambda i: (0, 0)),
            pl.BlockSpec((1, D), lambda i: (0, 0)),
            pl.BlockSpec((D, D), lambda i: (0, 0)),
            pl.BlockSpec((1, D), lambda i: (0, 0)),
        ],
        out_specs=pl.BlockSpec((E_BLK, D), lambda i: (i, 0)),
        out_shape=jax.ShapeDtypeStruct((N_EDGES, D), _f32),
    )(tmp_s, tmp_t, edge_features, We1e, b1e, We2, b2e)

    src3 = src.reshape(NW, N_CH, K)
    partials = _sc_scatter(zeros, src3, f)

    N_BLK = 2000
    out = pl.pallas_call(
        _node_mlp_body,
        grid=(N_NODES // N_BLK,),
        in_specs=[
            pl.BlockSpec((N_BLK, D), lambda i: (i, 0)),
            pl.BlockSpec((NC, N_BLK, D), lambda i: (0, i, 0)),
            pl.BlockSpec((D, D), lambda i: (0, 0)),
            pl.BlockSpec((D, D), lambda i: (0, 0)),
            pl.BlockSpec((1, D), lambda i: (0, 0)),
            pl.BlockSpec((D, D), lambda i: (0, 0)),
            pl.BlockSpec((1, D), lambda i: (0, 0)),
        ],
        out_specs=pl.BlockSpec((N_BLK, D), lambda i: (i, 0)),
        out_shape=jax.ShapeDtypeStruct((N_NODES, D), _f32),
    )(x, partials, Wn1x, Wn1a, b1n, Wn2, b2n)
    return out
